# Initial kernel scaffold; baseline (speedup 1.0000x reference)
import jax
import jax.numpy as jnp
from jax.experimental import pallas as pl


def kernel(token_ids, weight_float):
    raise NotImplementedError("write your pallas kernel here")


# speedup vs baseline: 14.1033x; 14.1033x over previous
import functools

import jax
import jax.numpy as jnp
from jax import lax
from jax.experimental import pallas as pl
from jax.experimental.pallas import tpu as pltpu
from jax.experimental.pallas import tpu_sc as plsc

EMBED = 16
BITS = 32
NUM_WORKERS = 32
CHUNK = 128


def _sc_gather(table, idx, n_tokens):
    b_per_w = n_tokens // NUM_WORKERS
    num_chunks = (b_per_w + CHUNK - 1) // CHUNK
    mesh = plsc.VectorSubcoreMesh(core_axis_name="c", subcore_axis_name="s")

    @functools.partial(
        pl.kernel,
        mesh=mesh,
        out_type=jax.ShapeDtypeStruct((n_tokens, EMBED), jnp.float32),
        scratch_types=[
            pltpu.VMEM((b_per_w,), jnp.int32),
            pltpu.VMEM((b_per_w, EMBED), jnp.float32),
            pltpu.SemaphoreType.DMA,
        ],
        compiler_params=pltpu.CompilerParams(use_tc_tiling_on_sc=False),
    )
    def k(table_hbm, idx_hbm, out_hbm, idx_v, rows_v, sem):
        wid = lax.axis_index("s") * 2 + lax.axis_index("c")
        base = wid * b_per_w
        pltpu.sync_copy(idx_hbm.at[pl.ds(base, b_per_w)], idx_v)
        copies = []
        off = 0
        for _ in range(num_chunks):
            n = min(CHUNK, b_per_w - off)
            c = pltpu.make_async_copy(
                table_hbm.at[idx_v.at[pl.ds(off, n)]],
                rows_v.at[pl.ds(off, n)],
                sem,
            )
            c.start()
            copies.append(c)
            off += n
        for c in copies:
            c.wait()
        pltpu.sync_copy(rows_v, out_hbm.at[pl.ds(base, b_per_w)])

    return k(table, idx)


def _expand_body(x_ref, o_ref):
    bn = x_ref.shape[0]
    bits = lax.bitcast_convert_type(x_ref[...], jnp.int32)
    parts = [
        jnp.broadcast_to(bits[:, e:e + 1], (bn, BITS)) for e in range(EMBED)
    ]
    big = jnp.concatenate(parts, axis=1)
    lane = lax.broadcasted_iota(jnp.int32, (bn, EMBED * BITS), 1)
    mask = jnp.left_shift(jnp.int32(1), 31 - (lane & (BITS - 1)))
    o_ref[...] = ((big & mask) != 0).astype(jnp.float32)


def _tc_expand(gathered, n_tokens):
    bn = 512
    grid = (n_tokens // bn,)
    return pl.pallas_call(
        _expand_body,
        grid=grid,
        in_specs=[pl.BlockSpec((bn, EMBED), lambda i: (i, 0))],
        out_specs=pl.BlockSpec((bn, EMBED * BITS), lambda i: (i, 0)),
        out_shape=jax.ShapeDtypeStruct((n_tokens, EMBED * BITS), jnp.float32),
    )(gathered)


def kernel(token_ids, weight_float):
    batch_shape = token_ids.shape
    flat_ids = token_ids.reshape(-1).astype(jnp.int32)
    n_tokens = flat_ids.shape[0]
    gathered = _sc_gather(weight_float, flat_ids, n_tokens)
    out = _tc_expand(gathered, n_tokens)
    return out.reshape(batch_shape + (EMBED, BITS))
